# Initial kernel scaffold; baseline (speedup 1.0000x reference)
import jax
import jax.numpy as jnp
from jax.experimental import pallas as pl


def kernel(xyz1, xyz2, points1, points2):
    raise NotImplementedError("write your pallas kernel here")


# speedup vs baseline: 23.3445x; 23.3445x over previous
import functools

import jax
import jax.numpy as jnp
from jax import lax
from jax.experimental import pallas as pl
from jax.experimental.pallas import tpu as pltpu
from jax.experimental.pallas import tpu_sc as plsc

B = 16
N1 = 8192
N2 = 2048
C1 = 128
C2 = 256
COUT = C2 + C1

QB = 256

NC = 2
NS = 16
NW = NC * NS
TOT = B * N1
QPW = TOT // NW
G = 32
NCH = QPW // G


def _tc_body(x1_ref, x2t_ref, idx_ref, w_ref):
    b = pl.program_id(0)
    x1 = x1_ref[0]
    x2t = x2t_ref[0]
    d0 = x1[:, 0:1] - x2t[0:1, :]
    d1 = x1[:, 1:2] - x2t[1:2, :]
    d2 = x1[:, 2:3] - x2t[2:3, :]
    sqd = d0 * d0 + d1 * d1 + d2 * d2
    iota = lax.broadcasted_iota(jnp.int32, (QB, N2), 1)
    cur = sqd
    idxs, dists = [], []
    for _ in range(3):
        m = jnp.min(cur, axis=1, keepdims=True)
        i = jnp.min(jnp.where(cur == m, iota, N2), axis=1, keepdims=True)
        cur = jnp.where(iota == i, jnp.inf, cur)
        idxs.append(i)
        dists.append(m)
    d = jnp.concatenate(dists, axis=1)
    d = jnp.maximum(d, 1e-10)
    r = 1.0 / d
    w = r / jnp.sum(r, axis=1, keepdims=True)
    idx = jnp.concatenate(idxs, axis=1) + b * N2
    idx_ref[0] = idx.astype(jnp.int32)
    w_ref[0] = w


@jax.jit
def _three_nn(xyz1, x2t):
    return pl.pallas_call(
        _tc_body,
        grid=(B, N1 // QB),
        in_specs=[
            pl.BlockSpec((1, QB, 3), lambda b, q: (b, q, 0)),
            pl.BlockSpec((1, 3, N2), lambda b, q: (b, 0, 0)),
        ],
        out_specs=[
            pl.BlockSpec((1, QB, 3), lambda b, q: (b, q, 0)),
            pl.BlockSpec((1, QB, 3), lambda b, q: (b, q, 0)),
        ],
        out_shape=[
            jax.ShapeDtypeStruct((B, N1, 3), jnp.int32),
            jax.ShapeDtypeStruct((B, N1, 3), jnp.float32),
        ],
    )(xyz1, x2t)


def _sc_body(table_hbm, idx_hbm, w_hbm, p1_hbm, out_hbm,
             idx_v, gath_v, w_v, o_v, p1_v, sem):
    wid = lax.axis_index("s") * NC + lax.axis_index("c")
    q0 = wid * QPW

    def chunk_body(ci, _):
        base = q0 + ci * G
        pltpu.sync_copy(idx_hbm.at[pl.ds(base * 3, 3 * G)], idx_v)
        pltpu.async_copy(table_hbm.at[idx_v], gath_v, sem).wait()
        pltpu.sync_copy(w_hbm.at[pl.ds(base * 3, 3 * G)], w_v.at[pl.ds(0, 3 * G)])
        pltpu.sync_copy(p1_hbm.at[pl.ds(base, G), :], p1_v)

        def q_body(qi, _):
            row = 3 * qi
            wv = w_v[pl.ds(row, 16)]
            w0 = wv[0]
            w1 = wv[1]
            w2 = wv[2]
            for j in range(C2 // 16):
                s = pl.ds(j * 16, 16)
                acc = (gath_v[row, s] * w0 + gath_v[row + 1, s] * w1
                       + gath_v[row + 2, s] * w2)
                o_v[qi, s] = acc
            return 0

        lax.fori_loop(0, G, q_body, 0)
        pltpu.sync_copy(o_v, out_hbm.at[pl.ds(base, G), pl.ds(0, C2)])
        pltpu.sync_copy(p1_v, out_hbm.at[pl.ds(base, G), pl.ds(C2, C1)])
        return 0

    lax.fori_loop(0, NCH, chunk_body, 0)


@jax.jit
def _interp(table, idx_flat, w_flat, p1_flat):
    mesh = plsc.VectorSubcoreMesh(core_axis_name="c", subcore_axis_name="s")
    f = functools.partial(
        pl.kernel,
        out_type=jax.ShapeDtypeStruct((TOT, COUT), jnp.float32),
        mesh=mesh,
        scratch_types=[
            pltpu.VMEM((3 * G,), jnp.int32),
            pltpu.VMEM((3 * G, C2), jnp.float32),
            pltpu.VMEM((3 * G + 16,), jnp.float32),
            pltpu.VMEM((G, C2), jnp.float32),
            pltpu.VMEM((G, C1), jnp.float32),
            pltpu.SemaphoreType.DMA,
        ],
    )(_sc_body)
    return f(table, idx_flat, w_flat, p1_flat)


def kernel(xyz1, xyz2, points1, points2):
    x2t = jnp.transpose(xyz2, (0, 2, 1))
    idx, w = _three_nn(xyz1, x2t)
    table = points2.reshape(B * N2, C2)
    out = _interp(table, idx.reshape(-1), w.reshape(-1),
                  points1.reshape(TOT, C1))
    return out.reshape(B, N1, 1, COUT)
